# baseline (speedup 1.0000x reference)
import jax
import jax.numpy as jnp
from jax import lax
from jax.experimental import pallas as pl


def _mm_body(u_ref, it_ref, o_ref):
  it = it_ref[...]
  n_chunks = 4
  rows = u_ref.shape[0] // n_chunks
  for k in range(n_chunks):
    acc = lax.dot_general(
        u_ref[k * rows:(k + 1) * rows, :], it,
        dimension_numbers=(((1,), (1,)), ((), ())),
        preferred_element_type=jnp.float32,
    )
    o_ref[k * rows:(k + 1) * rows, :] = acc.astype(jnp.bfloat16)


def _tc_scores(emb, batch, dim):
  out = pl.pallas_call(
      _mm_body,
      grid=(1,),
      in_specs=[
          pl.BlockSpec((batch, dim), lambda i: (0, 0)),
          pl.BlockSpec((batch, dim), lambda i: (1, 0)),
      ],
      out_specs=pl.BlockSpec((batch, batch), lambda i: (0, 0)),
      out_shape=jax.ShapeDtypeStruct((batch, batch), jnp.bfloat16),
  )(emb, emb)
  return out.astype(jnp.float32)


@jax.jit
def kernel(id_embedding, user_tensor, item_tensor):
  batch = user_tensor.shape[0]
  dim = id_embedding.shape[1]
  idx = jnp.concatenate(
      [user_tensor.astype(jnp.int32), item_tensor.astype(jnp.int32)])
  emb = jnp.take(id_embedding, idx, axis=0)
  return _tc_scores(emb, batch, dim)
